# baseline (speedup 1.0000x reference)
import functools

import jax
import jax.numpy as jnp
from jax import lax
from jax.experimental import pallas as pl
from jax.experimental.pallas import tpu as pltpu
from jax.experimental.pallas import tpu_sc as plsc

NN = 10000
EE = 320000
DD = 128

NC = 2
NS = 16
NW = NC * NS
LANES = 16

EPW = EE // NW
NPHASE = 5
EPP = EPW // NPHASE
CHUNK = 40
NCHUNK = EPP // CHUNK
DEPTH = 5
ROWS_A = 624
ROWS_B = NN - 15 * ROWS_A
NSUB = 3

assert NCHUNK % DEPTH == 0


def _matmul_body(v_ref, w_ref, o_ref):
    o_ref[...] = jax.lax.dot_general(
        v_ref[...], w_ref[...],
        dimension_numbers=(((1,), (1,)), ((), ())),
        preferred_element_type=jnp.float32,
    )


def _combine_body(p0_ref, p1_ref, o_ref):
    o_ref[...] = jnp.maximum(p0_ref[...] + p1_ref[...], 0.0)


def _sc_body(support_hbm, ei_hbm, w_hbm, zeros_hbm, out_hbm,
             acc, src_v, dst_v, w_v, rows, gsem, ssem):
    c = lax.axis_index("c")
    s = lax.axis_index("s")
    wid = s * NC + c

    @pl.when(s < NS - 1)
    def _():
        pltpu.sync_copy(zeros_hbm.at[pl.ds(0, ROWS_A)],
                        acc.at[pl.ds(s * ROWS_A, ROWS_A)])

    @pl.when(s == NS - 1)
    def _():
        pltpu.sync_copy(zeros_hbm,
                        acc.at[pl.ds((NS - 1) * ROWS_A, ROWS_B)])

    plsc.subcore_barrier()

    iota16 = lax.iota(jnp.int32, LANES)

    def issue_gather(j, b):
        pltpu.async_copy(
            support_hbm.at[src_v.at[pl.ds(j * CHUNK, CHUNK)]],
            rows[b].at[pl.ds(0, CHUNK)], gsem.at[b])

    def wait_gather(b):
        pltpu.make_async_copy(
            support_hbm.at[src_v.at[pl.ds(0, CHUNK)]],
            rows[b].at[pl.ds(0, CHUNK)], gsem.at[b]).wait()

    def wait_scatter(b):
        for _ in range(NSUB):
            pltpu.make_async_copy(
                rows[b].at[pl.ds(0, LANES)],
                acc.at[jnp.zeros((LANES,), jnp.int32)],
                ssem.at[b]).wait()

    def sub(j, a, nxt, guard, gather):
        wait_gather(a)
        if guard:
            wait_scatter(nxt)
        if gather:
            issue_gather(j + DEPTH - 1, nxt)
        ebase = j * CHUNK

        @plsc.parallel_loop(0, CHUNK, unroll=4)
        def _(e):
            bw = plsc.load_gather(
                w_v, [jnp.broadcast_to(ebase + e, (LANES,))])
            for k in range(DD // LANES):
                sl = pl.ds(k * LANES, LANES)
                rows[a][e, sl] = rows[a][e, sl] * bw

        for q in range(NSUB):
            idx = dst_v[pl.ds(ebase + q * LANES, LANES)]
            if (q + 1) * LANES > CHUNK:
                idx = jnp.where(iota16 < CHUNK - q * LANES, idx, NN)
            pltpu.async_copy(rows[a].at[pl.ds(q * LANES, LANES)],
                             acc.at[idx], ssem.at[a], add=True)

    for p in range(NPHASE):
        base = wid * EPW + p * EPP
        pltpu.sync_copy(ei_hbm.at[pl.ds(base, EPP)], src_v)
        pltpu.sync_copy(ei_hbm.at[pl.ds(EE + base, EPP)],
                        dst_v.at[pl.ds(0, EPP)])
        pltpu.sync_copy(w_hbm.at[pl.ds(base, EPP)], w_v)

        for b in range(DEPTH - 1):
            issue_gather(b, b)
        sub(0, 0, DEPTH - 1, guard=False, gather=True)

        @pl.loop(1, NCHUNK - DEPTH + 1, step=DEPTH)
        def _(j):
            for i in range(DEPTH):
                a = (1 + i) % DEPTH
                sub(j + i, a, (a + DEPTH - 1) % DEPTH,
                    guard=True, gather=True)

        for j in range(NCHUNK - DEPTH + 1, NCHUNK):
            a = j % DEPTH
            sub(j, a, (a + DEPTH - 1) % DEPTH, guard=False, gather=False)

        for i in range(DEPTH):
            wait_scatter((NCHUNK - DEPTH + i) % DEPTH)

    plsc.subcore_barrier()

    @pl.when(s < NS - 1)
    def _():
        pltpu.sync_copy(acc.at[pl.ds(s * ROWS_A, ROWS_A)],
                        out_hbm.at[pl.ds(c * NN + s * ROWS_A, ROWS_A)])

    @pl.when(s == NS - 1)
    def _():
        pltpu.sync_copy(acc.at[pl.ds((NS - 1) * ROWS_A, ROWS_B)],
                        out_hbm.at[pl.ds(c * NN + (NS - 1) * ROWS_A, ROWS_B)])


@functools.partial(
    pl.kernel,
    out_type=jax.ShapeDtypeStruct((2 * NN, DD), jnp.float32),
    mesh=plsc.VectorSubcoreMesh(core_axis_name="c", subcore_axis_name="s"),
    scratch_types=[
        pltpu.VMEM_SHARED((NN + 8, DD), jnp.float32),
        pltpu.VMEM((EPP,), jnp.int32),
        pltpu.VMEM((EPP + LANES,), jnp.int32),
        pltpu.VMEM((EPP,), jnp.float32),
    ] + [pltpu.VMEM((NSUB * LANES, DD), jnp.float32) for _ in range(DEPTH)] + [
        pltpu.SemaphoreType.DMA((DEPTH,)),
        pltpu.SemaphoreType.DMA((DEPTH,)),
    ],
    compiler_params=pltpu.CompilerParams(needs_layout_passes=False),
)
def _sc_spmm(support_hbm, ei_hbm, w_hbm, zeros_hbm, out_hbm,
             acc, src_v, dst_v, w_v, *rest):
    rows = rest[:DEPTH]
    gsem, ssem = rest[DEPTH], rest[DEPTH + 1]
    _sc_body(support_hbm, ei_hbm, w_hbm, zeros_hbm, out_hbm,
             acc, src_v, dst_v, w_v, rows, gsem, ssem)


def kernel(v, edge_index, edge_weight, W):
    support = pl.pallas_call(
        _matmul_body,
        grid=(10,),
        in_specs=[
            pl.BlockSpec((NN // 10, DD), lambda i: (i, 0)),
            pl.BlockSpec((DD, DD), lambda i: (0, 0)),
        ],
        out_specs=pl.BlockSpec((NN // 10, DD), lambda i: (i, 0)),
        out_shape=jax.ShapeDtypeStruct((NN, DD), jnp.float32),
    )(v, W)

    zeros = jnp.zeros((ROWS_B, DD), jnp.float32)

    partial = _sc_spmm(support, edge_index.reshape(2 * EE), edge_weight,
                       zeros)

    out = pl.pallas_call(
        _combine_body,
        grid=(10,),
        in_specs=[
            pl.BlockSpec((NN // 10, DD), lambda i: (i, 0)),
            pl.BlockSpec((NN // 10, DD), lambda i: (i + 10, 0)),
        ],
        out_specs=pl.BlockSpec((NN // 10, DD), lambda i: (i, 0)),
        out_shape=jax.ShapeDtypeStruct((NN, DD), jnp.float32),
    )(partial, partial)
    return out


# speedup vs baseline: 1.0291x; 1.0291x over previous
import functools

import jax
import jax.numpy as jnp
from jax import lax
from jax.experimental import pallas as pl
from jax.experimental.pallas import tpu as pltpu
from jax.experimental.pallas import tpu_sc as plsc

NN = 10000
EE = 320000
DD = 128

NC = 2
NS = 16
NW = NC * NS
LANES = 16

EPW = EE // NW
NPHASE = 5
EPP = EPW // NPHASE
CHUNK = 40
NCHUNK = EPP // CHUNK
DEPTH = 5
ROWS_A = 624
ROWS_B = NN - 15 * ROWS_A

assert NCHUNK % DEPTH == 0


def _matmul_body(v_ref, w_ref, o_ref):
    o_ref[...] = jax.lax.dot_general(
        v_ref[...], w_ref[...],
        dimension_numbers=(((1,), (1,)), ((), ())),
        preferred_element_type=jnp.float32,
    )


def _combine_body(p0_ref, p1_ref, o_ref):
    o_ref[...] = jnp.maximum(p0_ref[...] + p1_ref[...], 0.0)


def _sc_body(support_hbm, ei_hbm, w_hbm, zeros_hbm, out_hbm,
             acc, src_v, dst_v, w_v, rows, gsem, ssem):
    c = lax.axis_index("c")
    s = lax.axis_index("s")
    wid = s * NC + c

    @pl.when(s < NS - 1)
    def _():
        pltpu.sync_copy(zeros_hbm.at[pl.ds(0, ROWS_A)],
                        acc.at[pl.ds(s * ROWS_A, ROWS_A)])

    @pl.when(s == NS - 1)
    def _():
        pltpu.sync_copy(zeros_hbm,
                        acc.at[pl.ds((NS - 1) * ROWS_A, ROWS_B)])

    plsc.subcore_barrier()

    def issue_gather(j, b):
        pltpu.async_copy(
            support_hbm.at[src_v.at[j]],
            rows[b], gsem.at[b])

    def wait_gather(b):
        pltpu.make_async_copy(
            support_hbm.at[src_v.at[0]],
            rows[b], gsem.at[b]).wait()

    def wait_scatter(b):
        pltpu.make_async_copy(rows[b], acc.at[dst_v.at[0]], ssem.at[b]).wait()

    def sub(j, a, nxt, guard, gather):
        wait_gather(a)
        if guard:
            wait_scatter(nxt)
        if gather:
            issue_gather(j + DEPTH - 1, nxt)
        ebase = j * CHUNK

        @plsc.parallel_loop(0, CHUNK, unroll=4)
        def _(e):
            bw = plsc.load_gather(
                w_v, [jnp.broadcast_to(ebase + e, (LANES,))])
            for k in range(DD // LANES):
                sl = pl.ds(k * LANES, LANES)
                rows[a][e, sl] = rows[a][e, sl] * bw

        pltpu.async_copy(rows[a], acc.at[dst_v.at[j]], ssem.at[a], add=True)

    for p in range(NPHASE):
        base = wid * EPW + p * EPP
        pltpu.sync_copy(ei_hbm.at[0, wid * NPHASE + p], src_v)
        pltpu.sync_copy(ei_hbm.at[1, wid * NPHASE + p], dst_v)
        pltpu.sync_copy(w_hbm.at[pl.ds(base, EPP)], w_v)

        for b in range(DEPTH - 1):
            issue_gather(b, b)
        sub(0, 0, DEPTH - 1, guard=False, gather=True)

        @pl.loop(1, NCHUNK - DEPTH + 1, step=DEPTH)
        def _(j):
            for i in range(DEPTH):
                a = (1 + i) % DEPTH
                sub(j + i, a, (a + DEPTH - 1) % DEPTH,
                    guard=True, gather=True)

        for j in range(NCHUNK - DEPTH + 1, NCHUNK):
            a = j % DEPTH
            sub(j, a, (a + DEPTH - 1) % DEPTH, guard=False, gather=False)

        for i in range(DEPTH):
            wait_scatter((NCHUNK - DEPTH + i) % DEPTH)

    plsc.subcore_barrier()

    @pl.when(s < NS - 1)
    def _():
        pltpu.sync_copy(acc.at[pl.ds(s * ROWS_A, ROWS_A)],
                        out_hbm.at[pl.ds(c * NN + s * ROWS_A, ROWS_A)])

    @pl.when(s == NS - 1)
    def _():
        pltpu.sync_copy(acc.at[pl.ds((NS - 1) * ROWS_A, ROWS_B)],
                        out_hbm.at[pl.ds(c * NN + (NS - 1) * ROWS_A, ROWS_B)])


@functools.partial(
    pl.kernel,
    out_type=jax.ShapeDtypeStruct((2 * NN, DD), jnp.float32),
    mesh=plsc.VectorSubcoreMesh(core_axis_name="c", subcore_axis_name="s"),
    scratch_types=[
        pltpu.VMEM_SHARED((NN, DD), jnp.float32),
        pltpu.VMEM((NCHUNK, CHUNK), jnp.int32),
        pltpu.VMEM((NCHUNK, CHUNK), jnp.int32),
        pltpu.VMEM((EPP,), jnp.float32),
    ] + [pltpu.VMEM((CHUNK, DD), jnp.float32) for _ in range(DEPTH)] + [
        pltpu.SemaphoreType.DMA((DEPTH,)),
        pltpu.SemaphoreType.DMA((DEPTH,)),
    ],
    compiler_params=pltpu.CompilerParams(needs_layout_passes=False),
)
def _sc_spmm(support_hbm, ei_hbm, w_hbm, zeros_hbm, out_hbm,
             acc, src_v, dst_v, w_v, *rest):
    rows = rest[:DEPTH]
    gsem, ssem = rest[DEPTH], rest[DEPTH + 1]
    _sc_body(support_hbm, ei_hbm, w_hbm, zeros_hbm, out_hbm,
             acc, src_v, dst_v, w_v, rows, gsem, ssem)


def kernel(v, edge_index, edge_weight, W):
    support = pl.pallas_call(
        _matmul_body,
        grid=(5,),
        in_specs=[
            pl.BlockSpec((NN // 5, DD), lambda i: (i, 0)),
            pl.BlockSpec((DD, DD), lambda i: (0, 0)),
        ],
        out_specs=pl.BlockSpec((NN // 5, DD), lambda i: (i, 0)),
        out_shape=jax.ShapeDtypeStruct((NN, DD), jnp.float32),
    )(v, W)

    ei = edge_index.reshape(2, NW * NPHASE, NCHUNK, CHUNK)
    zeros = jnp.zeros((ROWS_B, DD), jnp.float32)

    partial = _sc_spmm(support, ei, edge_weight, zeros)

    out = pl.pallas_call(
        _combine_body,
        grid=(5,),
        in_specs=[
            pl.BlockSpec((NN // 5, DD), lambda i: (i, 0)),
            pl.BlockSpec((NN // 5, DD), lambda i: (i + 5, 0)),
        ],
        out_specs=pl.BlockSpec((NN // 5, DD), lambda i: (i, 0)),
        out_shape=jax.ShapeDtypeStruct((NN, DD), jnp.float32),
    )(partial, partial)
    return out


# speedup vs baseline: 1.0507x; 1.0210x over previous
import functools

import jax
import jax.numpy as jnp
from jax import lax
from jax.experimental import pallas as pl
from jax.experimental.pallas import tpu as pltpu
from jax.experimental.pallas import tpu_sc as plsc

NN = 10000
EE = 320000
DD = 128

NC = 2
NS = 16
NW = NC * NS
LANES = 16

EPW = EE // NW
NPHASE = 5
EPP = EPW // NPHASE
CHUNK = 40
NCHUNK = EPP // CHUNK
DEPTH = 5
ROWS_A = 624
ROWS_B = NN - 15 * ROWS_A

assert NCHUNK % DEPTH == 0


def _matmul_body(v_ref, w_ref, o_ref):
    o_ref[...] = jax.lax.dot_general(
        v_ref[...], w_ref[...],
        dimension_numbers=(((1,), (1,)), ((), ())),
        preferred_element_type=jnp.float32,
    )


def _combine_body(p0_ref, p1_ref, o_ref):
    o_ref[...] = jnp.maximum(p0_ref[...] + p1_ref[...], 0.0)


def _sc_body(support_hbm, ei_hbm, w_hbm, zeros_hbm, out_hbm,
             acc, src_v, dst_v, w_v, rows, gsem, ssem):
    c = lax.axis_index("c")
    s = lax.axis_index("s")
    wid = s * NC + c

    @pl.when(s < NS - 1)
    def _():
        pltpu.sync_copy(zeros_hbm.at[pl.ds(0, ROWS_A)],
                        acc.at[pl.ds(s * ROWS_A, ROWS_A)])

    @pl.when(s == NS - 1)
    def _():
        pltpu.sync_copy(zeros_hbm,
                        acc.at[pl.ds((NS - 1) * ROWS_A, ROWS_B)])

    plsc.subcore_barrier()

    def issue_gather(j, b):
        pltpu.async_copy(
            support_hbm.at[src_v.at[j]],
            rows[b], gsem.at[b])

    def wait_gather(b):
        pltpu.make_async_copy(
            support_hbm.at[src_v.at[0]],
            rows[b], gsem.at[b]).wait()

    def wait_scatter(b):
        pltpu.make_async_copy(rows[b], acc.at[dst_v.at[0]], ssem.at[b]).wait()

    def sub(j, a, nxt, guard, gather):
        wait_gather(a)
        if guard:
            wait_scatter(nxt)
        if gather:
            issue_gather(j + DEPTH - 1, nxt)
        ebase = j * CHUNK

        @plsc.parallel_loop(0, CHUNK, unroll=4)
        def _(e):
            bw = plsc.load_gather(
                w_v, [jnp.broadcast_to(ebase + e, (LANES,))])
            for k in range(DD // LANES):
                sl = pl.ds(k * LANES, LANES)
                rows[a][e, sl] = rows[a][e, sl] * bw

        pltpu.async_copy(rows[a], acc.at[dst_v.at[j]], ssem.at[a], add=True)

    for p in range(NPHASE):
        base = wid * EPW + p * EPP
        pltpu.sync_copy(ei_hbm.at[0, wid * NPHASE + p], src_v)
        pltpu.sync_copy(ei_hbm.at[1, wid * NPHASE + p], dst_v)
        pltpu.sync_copy(w_hbm.at[pl.ds(base, EPP)], w_v)

        for b in range(DEPTH - 1):
            issue_gather(b, b)
        sub(0, 0, DEPTH - 1, guard=False, gather=True)

        @pl.loop(1, NCHUNK - DEPTH + 1, step=DEPTH)
        def _(j):
            for i in range(DEPTH):
                a = (1 + i) % DEPTH
                sub(j + i, a, (a + DEPTH - 1) % DEPTH,
                    guard=True, gather=True)

        for j in range(NCHUNK - DEPTH + 1, NCHUNK):
            a = j % DEPTH
            sub(j, a, (a + DEPTH - 1) % DEPTH, guard=False, gather=False)

        for i in range(DEPTH):
            wait_scatter((NCHUNK - DEPTH + i) % DEPTH)

    plsc.subcore_barrier()

    @pl.when(s < NS - 1)
    def _():
        pltpu.sync_copy(acc.at[pl.ds(s * ROWS_A, ROWS_A)],
                        out_hbm.at[pl.ds(c * NN + s * ROWS_A, ROWS_A)])

    @pl.when(s == NS - 1)
    def _():
        pltpu.sync_copy(acc.at[pl.ds((NS - 1) * ROWS_A, ROWS_B)],
                        out_hbm.at[pl.ds(c * NN + (NS - 1) * ROWS_A, ROWS_B)])


@functools.partial(
    pl.kernel,
    out_type=jax.ShapeDtypeStruct((2 * NN, DD), jnp.float32),
    mesh=plsc.VectorSubcoreMesh(core_axis_name="c", subcore_axis_name="s"),
    scratch_types=[
        pltpu.VMEM_SHARED((NN, DD), jnp.float32),
        pltpu.VMEM((NCHUNK, CHUNK), jnp.int32),
        pltpu.VMEM((NCHUNK, CHUNK), jnp.int32),
        pltpu.VMEM((EPP,), jnp.float32),
    ] + [pltpu.VMEM((CHUNK, DD), jnp.float32) for _ in range(DEPTH)] + [
        pltpu.SemaphoreType.DMA((DEPTH,)),
        pltpu.SemaphoreType.DMA((DEPTH,)),
    ],
    compiler_params=pltpu.CompilerParams(needs_layout_passes=False),
)
def _sc_spmm(support_hbm, ei_hbm, w_hbm, zeros_hbm, out_hbm,
             acc, src_v, dst_v, w_v, *rest):
    rows = rest[:DEPTH]
    gsem, ssem = rest[DEPTH], rest[DEPTH + 1]
    _sc_body(support_hbm, ei_hbm, w_hbm, zeros_hbm, out_hbm,
             acc, src_v, dst_v, w_v, rows, gsem, ssem)


def kernel(v, edge_index, edge_weight, W):
    support = pl.pallas_call(
        _matmul_body,
        grid=(2,),
        in_specs=[
            pl.BlockSpec((NN // 2, DD), lambda i: (i, 0)),
            pl.BlockSpec((DD, DD), lambda i: (0, 0)),
        ],
        out_specs=pl.BlockSpec((NN // 2, DD), lambda i: (i, 0)),
        out_shape=jax.ShapeDtypeStruct((NN, DD), jnp.float32),
    )(v, W)

    ei = edge_index.reshape(2, NW * NPHASE, NCHUNK, CHUNK)
    zeros = jnp.zeros((ROWS_B, DD), jnp.float32)

    partial = _sc_spmm(support, ei, edge_weight, zeros)

    out = pl.pallas_call(
        _combine_body,
        grid=(2,),
        in_specs=[
            pl.BlockSpec((NN // 2, DD), lambda i: (i, 0)),
            pl.BlockSpec((NN // 2, DD), lambda i: (i + 2, 0)),
        ],
        out_specs=pl.BlockSpec((NN // 2, DD), lambda i: (i, 0)),
        out_shape=jax.ShapeDtypeStruct((NN, DD), jnp.float32),
    )(partial, partial)
    return out
